# baseline (speedup 1.0000x reference)
import numpy as np
import jax
import jax.numpy as jnp
from jax import lax
from jax.experimental import pallas as pl
from jax.experimental.pallas import tpu as pltpu
from jax.experimental.pallas import tpu_sc as plsc

_B, _L, _D, _V = 16384, 200, 16, 100000
_N = _B * _L
_NC, _NS = 2, 16
_NW = _NC * _NS
_BPW = _B // _NW
_NG = _BPW // 16

_T = np.arange(16)[None, :]
_K = np.arange(16)[:, None]
_DK = (_T + _K) % 16


def _pe_rot_table():
    pos = np.arange(_L, dtype=np.float32)[:, None]
    i = np.arange(_D, dtype=np.float32)[None, :]
    angle = pos / np.power(10000.0, (2.0 * np.floor(i / 2.0)) / _D)
    pe = np.zeros((_L, _D), dtype=np.float32)
    pe[:, 0::2] = np.sin(angle[:, 0::2])
    pe[:, 1::2] = np.cos(angle[:, 1::2])
    return np.ascontiguousarray(pe[:, _DK]).reshape(-1)


def _idx_tables():
    colrot = _DK.astype(np.int32)
    dovec = ((_DK // 8) * 4096 + (_DK % 8) * 128 + _T).astype(np.int32)
    return np.concatenate([colrot.reshape(-1), dovec.reshape(-1)])


_PE_ROT = _pe_rot_table()
_IDX_TAB = _idx_tables()


def _rsqrt(a):
    bits = lax.bitcast_convert_type(a, jnp.int32)
    y = lax.bitcast_convert_type(
        jnp.int32(0x5F3759DF) - lax.shift_right_arithmetic(bits, 1),
        jnp.float32)
    for _ in range(2):
        y = y * (1.5 - 0.5 * a * y * y)
    return y


def _tree_sum(vs):
    vs = list(vs)
    while len(vs) > 1:
        nxt = [vs[i] + vs[i + 1] for i in range(0, len(vs) - 1, 2)]
        if len(vs) % 2:
            nxt.append(vs[-1])
        vs = nxt
    return vs[0]


def _tec_body(stamp_hbm, pe_hbm, it_hbm, gb_hbm, table_hbm, out_hbm,
              idx_v, rows_v, out_v, pe_v, it_v, gb_v, isem, gsem, osem):
    wid = lax.axis_index("s") * _NC + lax.axis_index("c")
    pltpu.sync_copy(pe_hbm, pe_v)
    pltpu.sync_copy(it_hbm, it_v)
    pltpu.sync_copy(gb_hbm, gb_v)
    b0 = wid * _BPW
    iota = lax.iota(jnp.int32, 16)
    grot = [gb_v[pl.ds(k * 16, 16)] for k in range(_D)]
    brot = [gb_v[pl.ds(256 + k * 16, 16)] for k in range(_D)]

    def start_idx(l, b):
        off = l * _B + b0
        pltpu.async_copy(stamp_hbm.at[pl.ds(off, _BPW)], idx_v[b], isem[b])

    def wait_idx(b):
        pltpu.make_async_copy(stamp_hbm.at[pl.ds(0, _BPW)], idx_v[b],
                              isem[b]).wait()

    def start_gather(b):
        pltpu.async_copy(table_hbm.at[idx_v[b]], rows_v[b], gsem[b])

    def wait_gather(b):
        pltpu.make_async_copy(table_hbm.at[idx_v[b]], rows_v[b],
                              gsem[b]).wait()

    def start_out(l, b):
        for i in range(2):
            off = (l * 256 + i * 128 + wid * 4) * 1024
            pltpu.async_copy(out_v[b].at[pl.ds(i * 4096, 4096)],
                             out_hbm.at[pl.ds(off, 4096)], osem[b])

    def wait_out(b):
        for i in range(2):
            pltpu.make_async_copy(out_v[b].at[pl.ds(i * 4096, 4096)],
                                  out_hbm.at[pl.ds(0, 4096)], osem[b]).wait()

    def compute(l, rows, out):
        lbase = l * 256

        def one_group(g):
            rowidx = g * 16 + iota
            gpart = (g // 8) * 1024 + (g % 8) * 16
            xs = []
            for k in range(_D):
                colv = it_v[pl.ds(k * 16, 16)]
                e = plsc.load_gather(rows, [rowidx, colv])
                xs.append(e + pe_v[pl.ds(lbase + k * 16, 16)])
            mu = _tree_sum(xs) * (1.0 / _D)
            devs = [x - mu for x in xs]
            s2 = _tree_sum([dv * dv for dv in devs])
            r = _rsqrt(s2 * (1.0 / _D) + 1e-5)
            for k in range(_D):
                sidx = it_v[pl.ds(256 + k * 16, 16)] + gpart
                plsc.store_scatter(out, [sidx],
                                   devs[k] * (r * grot[k]) + brot[k])

        def group_body(h, carry):
            one_group(2 * h)
            one_group(2 * h + 1)
            return carry

        lax.fori_loop(0, _NG // 2, group_body, 0)

    start_idx(0, 0)
    start_idx(1, 1)
    wait_idx(0)
    start_gather(0)

    def outer(i, carry):
        for b in range(2):
            c = 2 * i + b

            @pl.when(c + 1 < _L)
            def _():
                wait_idx(b ^ 1)
                start_gather(b ^ 1)

            wait_gather(b)

            @pl.when(c >= 2)
            def _():
                wait_out(b)

            compute(c, rows_v[b], out_v[b])
            start_out(c, b)

            @pl.when(c + 2 < _L)
            def _():
                start_idx(c + 2, b)
        return carry

    lax.fori_loop(0, _L // 2, outer, 0)
    wait_out(0)
    wait_out(1)


def kernel(stamp, table, ln_gamma, ln_beta):
    stamp_t = jnp.transpose(stamp).reshape(-1)
    pe_const = jnp.asarray(_PE_ROT)
    it_const = jnp.asarray(_IDX_TAB)
    dk = jnp.asarray(_DK.reshape(-1))
    gb_const = jnp.concatenate([ln_gamma[dk], ln_beta[dk]])
    mesh = plsc.VectorSubcoreMesh(core_axis_name="c", subcore_axis_name="s")
    run = pl.kernel(
        _tec_body,
        compiler_params=pltpu.CompilerParams(
            needs_layout_passes=False, use_tc_tiling_on_sc=False),
        out_type=jax.ShapeDtypeStruct((_N * _D,), jnp.float32),
        mesh=mesh,
        scratch_types=[
            [pltpu.VMEM((_BPW,), jnp.int32)] * 2,
            [pltpu.VMEM((_BPW, _D), jnp.float32)] * 2,
            [pltpu.VMEM((_BPW * _D,), jnp.float32)] * 2,
            pltpu.VMEM((_L * _D * 16,), jnp.float32),
            pltpu.VMEM((512,), jnp.int32),
            pltpu.VMEM((512,), jnp.float32),
            [pltpu.SemaphoreType.DMA] * 2,
            [pltpu.SemaphoreType.DMA] * 2,
            [pltpu.SemaphoreType.DMA] * 2,
        ],
    )
    out = run(stamp_t, pe_const, it_const, gb_const, table)
    out = out.reshape(_L, 2, _B // 128, 8, 128)
    return jnp.transpose(out, (2, 4, 0, 1, 3)).reshape(_B, _L, _D)


# speedup vs baseline: 1.0869x; 1.0869x over previous
import numpy as np
import jax
import jax.numpy as jnp
from jax import lax
from jax.experimental import pallas as pl
from jax.experimental.pallas import tpu as pltpu
from jax.experimental.pallas import tpu_sc as plsc

_B, _L, _D, _V = 16384, 200, 16, 100000
_N = _B * _L
_NC, _NS = 2, 16
_NW = _NC * _NS
_BPW = _B // _NW
_NG = _BPW // 16


def _pe_lane_table():
    pos = np.arange(_L, dtype=np.float32)[:, None]
    i = np.arange(_D, dtype=np.float32)[None, :]
    angle = pos / np.power(10000.0, (2.0 * np.floor(i / 2.0)) / _D)
    pe = np.zeros((_L, _D), dtype=np.float32)
    pe[:, 0::2] = np.sin(angle[:, 0::2])
    pe[:, 1::2] = np.cos(angle[:, 1::2])
    return np.ascontiguousarray(
        np.broadcast_to(pe[:, :, None], (_L, _D, 16))).reshape(-1)


_PE_LANE = _pe_lane_table()


def _rsqrt(a):
    bits = lax.bitcast_convert_type(a, jnp.int32)
    y = lax.bitcast_convert_type(
        jnp.int32(0x5F3759DF) - lax.shift_right_arithmetic(bits, 1),
        jnp.float32)
    for _ in range(2):
        y = y * (1.5 - 0.5 * a * y * y)
    return y


def _tree_sum(vs):
    vs = list(vs)
    while len(vs) > 1:
        nxt = [vs[i] + vs[i + 1] for i in range(0, len(vs) - 1, 2)]
        if len(vs) % 2:
            nxt.append(vs[-1])
        vs = nxt
    return vs[0]


def _tec_body(stamp_hbm, pe_hbm, table_hbm, gamma_hbm, beta_hbm, out_hbm,
              idx_v, rows_v, out_v, pe_v, g_v, b_v, isem, gsem, osem):
    wid = lax.axis_index("s") * _NC + lax.axis_index("c")
    pltpu.sync_copy(pe_hbm, pe_v)
    pltpu.sync_copy(gamma_hbm, g_v)
    pltpu.sync_copy(beta_hbm, b_v)
    b0 = wid * _BPW
    iota = lax.iota(jnp.int32, 16)
    gsp = [plsc.load_gather(g_v, [jnp.full((16,), d, jnp.int32)])
           for d in range(_D)]
    bsp = [plsc.load_gather(b_v, [jnp.full((16,), d, jnp.int32)])
           for d in range(_D)]

    def start_idx(l, b):
        off = l * _B + b0
        pltpu.async_copy(stamp_hbm.at[pl.ds(off, _BPW)], idx_v[b], isem[b])

    def wait_idx(b):
        pltpu.make_async_copy(stamp_hbm.at[pl.ds(0, _BPW)], idx_v[b],
                              isem[b]).wait()

    def start_gather(b):
        pltpu.async_copy(table_hbm.at[idx_v[b]], rows_v[b], gsem[b])

    def wait_gather(b):
        pltpu.make_async_copy(table_hbm.at[idx_v[b]], rows_v[b],
                              gsem[b]).wait()

    def start_out(l, b):
        for i in range(2):
            off = (l * 256 + i * 128 + wid * 4) * 1024
            pltpu.async_copy(out_v[b].at[pl.ds(i * 4096, 4096)],
                             out_hbm.at[pl.ds(off, 4096)], osem[b])

    def wait_out(b):
        for i in range(2):
            pltpu.make_async_copy(out_v[b].at[pl.ds(i * 4096, 4096)],
                                  out_hbm.at[pl.ds(0, 4096)], osem[b]).wait()

    def compute(l, rows, out):
        lbase = l * 256
        psp = [pe_v[pl.ds(lbase + d * 16, 16)] for d in range(_D)]

        def one_group(g):
            rowidx = g * 16 + iota
            gpart = (g // 8) * 1024 + (g % 8) * 16
            xs = []
            for d in range(_D):
                e = plsc.load_gather(rows, [rowidx,
                                            jnp.full((16,), d, jnp.int32)])
                xs.append(e + psp[d])
            mu = _tree_sum(xs) * (1.0 / _D)
            devs = [x - mu for x in xs]
            s2 = _tree_sum([dv * dv for dv in devs])
            r = _rsqrt(s2 * (1.0 / _D) + 1e-5)
            for d in range(_D):
                obase = gpart + (d // 8) * 4096 + (d % 8) * 128
                plsc.store_scatter(out, [obase + iota],
                                   devs[d] * (r * gsp[d]) + bsp[d])

        @plsc.parallel_loop(0, _NG, 1, unroll=2)
        def group_body(g):
            one_group(g)

    start_idx(0, 0)
    start_idx(1, 1)
    wait_idx(0)
    start_gather(0)

    def outer(i, carry):
        for b in range(2):
            c = 2 * i + b

            @pl.when(c + 1 < _L)
            def _():
                wait_idx(b ^ 1)
                start_gather(b ^ 1)

            wait_gather(b)

            @pl.when(c >= 2)
            def _():
                wait_out(b)

            compute(c, rows_v[b], out_v[b])
            start_out(c, b)

            @pl.when(c + 2 < _L)
            def _():
                start_idx(c + 2, b)
        return carry

    lax.fori_loop(0, _L // 2, outer, 0)
    wait_out(0)
    wait_out(1)


def kernel(stamp, table, ln_gamma, ln_beta):
    stamp_t = jnp.transpose(stamp).reshape(-1)
    pe_const = jnp.asarray(_PE_LANE)
    mesh = plsc.VectorSubcoreMesh(core_axis_name="c", subcore_axis_name="s")
    run = pl.kernel(
        _tec_body,
        compiler_params=pltpu.CompilerParams(
            needs_layout_passes=False, use_tc_tiling_on_sc=False),
        out_type=jax.ShapeDtypeStruct((_N * _D,), jnp.float32),
        mesh=mesh,
        scratch_types=[
            [pltpu.VMEM((_BPW,), jnp.int32)] * 2,
            [pltpu.VMEM((_BPW, _D), jnp.float32)] * 2,
            [pltpu.VMEM((_BPW * _D,), jnp.float32)] * 2,
            pltpu.VMEM((_L * _D * 16,), jnp.float32),
            pltpu.VMEM((_D,), jnp.float32),
            pltpu.VMEM((_D,), jnp.float32),
            [pltpu.SemaphoreType.DMA] * 2,
            [pltpu.SemaphoreType.DMA] * 2,
            [pltpu.SemaphoreType.DMA] * 2,
        ],
    )
    out = run(stamp_t, pe_const, table, ln_gamma, ln_beta)
    out = out.reshape(_L, 2, _B // 128, 8, 128)
    return jnp.transpose(out, (2, 4, 0, 1, 3)).reshape(_B, _L, _D)


# speedup vs baseline: 1.4312x; 1.3167x over previous
import numpy as np
import jax
import jax.numpy as jnp
from jax import lax
from jax.experimental import pallas as pl
from jax.experimental.pallas import tpu as pltpu
from jax.experimental.pallas import tpu_sc as plsc

_B, _L, _D, _V = 16384, 200, 16, 100000
_N = _B * _L
_NC, _NS = 2, 16
_NW = _NC * _NS
_BPW = _B // _NW
_NG = _BPW // 16


def _pe_lane_table():
    pos = np.arange(_L, dtype=np.float32)[:, None]
    i = np.arange(_D, dtype=np.float32)[None, :]
    angle = pos / np.power(10000.0, (2.0 * np.floor(i / 2.0)) / _D)
    pe = np.zeros((_L, _D), dtype=np.float32)
    pe[:, 0::2] = np.sin(angle[:, 0::2])
    pe[:, 1::2] = np.cos(angle[:, 1::2])
    return np.ascontiguousarray(
        np.broadcast_to(pe[:, :, None], (_L, _D, 16))).reshape(-1)


_PE_LANE = _pe_lane_table()


def _rsqrt(a):
    bits = lax.bitcast_convert_type(a, jnp.int32)
    y = lax.bitcast_convert_type(
        jnp.int32(0x5F3759DF) - lax.shift_right_arithmetic(bits, 1),
        jnp.float32)
    for _ in range(2):
        y = y * (1.5 - 0.5 * a * y * y)
    return y


def _tree_sum(vs):
    vs = list(vs)
    while len(vs) > 1:
        nxt = [vs[i] + vs[i + 1] for i in range(0, len(vs) - 1, 2)]
        if len(vs) % 2:
            nxt.append(vs[-1])
        vs = nxt
    return vs[0]


def _tec_body(stamp_hbm, pe_hbm, table_hbm, gamma_hbm, beta_hbm, out_hbm,
              idx_v, rows_v, out_v, pe_v, g_v, b_v, isem, gsem, osem):
    wid = lax.axis_index("s") * _NC + lax.axis_index("c")
    pltpu.sync_copy(pe_hbm, pe_v)
    pltpu.sync_copy(gamma_hbm, g_v)
    pltpu.sync_copy(beta_hbm, b_v)
    b0 = wid * _BPW
    iota = lax.iota(jnp.int32, 16)
    gsp = [plsc.load_gather(g_v, [jnp.full((16,), d, jnp.int32)])
           for d in range(_D)]
    bsp = [plsc.load_gather(b_v, [jnp.full((16,), d, jnp.int32)])
           for d in range(_D)]

    def start_idx(l, b):
        off = l * _B + b0
        pltpu.async_copy(stamp_hbm.at[pl.ds(off, _BPW)], idx_v[b], isem[b])

    def wait_idx(b):
        pltpu.make_async_copy(stamp_hbm.at[pl.ds(0, _BPW)], idx_v[b],
                              isem[b]).wait()

    def start_gather(b):
        pltpu.async_copy(table_hbm.at[idx_v[b]], rows_v[b], gsem[b])

    def wait_gather(b):
        pltpu.make_async_copy(table_hbm.at[idx_v[b]], rows_v[b],
                              gsem[b]).wait()

    def start_out(l, b):
        for i in range(2):
            off = (l * 256 + i * 128 + wid * 4) * 1024
            pltpu.async_copy(out_v[b].at[pl.ds(i * 4096, 4096)],
                             out_hbm.at[pl.ds(off, 4096)], osem[b])

    def wait_out(b):
        for i in range(2):
            pltpu.make_async_copy(out_v[b].at[pl.ds(i * 4096, 4096)],
                                  out_hbm.at[pl.ds(0, 4096)], osem[b]).wait()

    def compute(l, rows, out):
        lbase = l * 256
        psp = [pe_v[pl.ds(lbase + d * 16, 16)] for d in range(_D)]

        def one_group(g):
            rowidx = g * 16 + iota
            gpart = (g // 8) * 1024 + (g % 8) * 16
            xs = []
            for d in range(_D):
                e = plsc.load_gather(rows, [rowidx,
                                            jnp.full((16,), d, jnp.int32)])
                xs.append(e + psp[d])
            mu = _tree_sum(xs) * (1.0 / _D)
            devs = [x - mu for x in xs]
            s2 = _tree_sum([dv * dv for dv in devs])
            r = _rsqrt(s2 * (1.0 / _D) + 1e-5)
            for d in range(_D):
                obase = gpart + (d // 8) * 4096 + (d % 8) * 128
                plsc.store_scatter(out, [obase + iota],
                                   devs[d] * (r * gsp[d]) + bsp[d])

        def group_body(g, carry):
            one_group(g)
            return carry

        lax.fori_loop(0, _NG, group_body, 0)

    start_idx(0, 0)
    start_idx(1, 1)
    wait_idx(0)
    start_gather(0)

    def outer(i, carry):
        for b in range(2):
            c = 2 * i + b

            @pl.when(c + 1 < _L)
            def _():
                wait_idx(b ^ 1)
                start_gather(b ^ 1)

            wait_gather(b)

            @pl.when(c >= 2)
            def _():
                wait_out(b)

            compute(c, rows_v[b], out_v[b])
            start_out(c, b)

            @pl.when(c + 2 < _L)
            def _():
                start_idx(c + 2, b)
        return carry

    lax.fori_loop(0, _L // 2, outer, 0)
    wait_out(0)
    wait_out(1)


def kernel(stamp, table, ln_gamma, ln_beta):
    stamp_t = jnp.transpose(stamp).reshape(-1)
    pe_const = jnp.asarray(_PE_LANE)
    mesh = plsc.VectorSubcoreMesh(core_axis_name="c", subcore_axis_name="s")
    run = pl.kernel(
        _tec_body,
        compiler_params=pltpu.CompilerParams(
            needs_layout_passes=False, use_tc_tiling_on_sc=False),
        out_type=jax.ShapeDtypeStruct((_N * _D,), jnp.float32),
        mesh=mesh,
        scratch_types=[
            [pltpu.VMEM((_BPW,), jnp.int32)] * 2,
            [pltpu.VMEM((_BPW, _D), jnp.float32)] * 2,
            [pltpu.VMEM((_BPW * _D,), jnp.float32)] * 2,
            pltpu.VMEM((_L * _D * 16,), jnp.float32),
            pltpu.VMEM((_D,), jnp.float32),
            pltpu.VMEM((_D,), jnp.float32),
            [pltpu.SemaphoreType.DMA] * 2,
            [pltpu.SemaphoreType.DMA] * 2,
            [pltpu.SemaphoreType.DMA] * 2,
        ],
    )
    out = run(stamp_t, pe_const, table, ln_gamma, ln_beta)
    out = out.reshape(_L, 2, _B // 128, 8, 128)
    return jnp.transpose(out, (2, 4, 0, 1, 3)).reshape(_B, _L, _D)


# speedup vs baseline: 1.6071x; 1.1229x over previous
import numpy as np
import jax
import jax.numpy as jnp
from jax import lax
from jax.experimental import pallas as pl
from jax.experimental.pallas import tpu as pltpu
from jax.experimental.pallas import tpu_sc as plsc

_B, _L, _D, _V = 16384, 200, 16, 100000
_N = _B * _L
_NC, _NS = 2, 16
_NW = _NC * _NS
_BPW = _B // _NW
_NG = _BPW // 16


def _pe_lane_table():
    pos = np.arange(_L, dtype=np.float32)[:, None]
    i = np.arange(_D, dtype=np.float32)[None, :]
    angle = pos / np.power(10000.0, (2.0 * np.floor(i / 2.0)) / _D)
    pe = np.zeros((_L, _D), dtype=np.float32)
    pe[:, 0::2] = np.sin(angle[:, 0::2])
    pe[:, 1::2] = np.cos(angle[:, 1::2])
    return np.ascontiguousarray(
        np.broadcast_to(pe[:, :, None], (_L, _D, 16))).reshape(-1)


_PE_LANE = _pe_lane_table()


def _rsqrt(a):
    bits = lax.bitcast_convert_type(a, jnp.int32)
    y = lax.bitcast_convert_type(
        jnp.int32(0x5F3759DF) - lax.shift_right_arithmetic(bits, 1),
        jnp.float32)
    for _ in range(2):
        y = y * (1.5 - 0.5 * a * y * y)
    return y


def _tree_sum(vs):
    vs = list(vs)
    while len(vs) > 1:
        nxt = [vs[i] + vs[i + 1] for i in range(0, len(vs) - 1, 2)]
        if len(vs) % 2:
            nxt.append(vs[-1])
        vs = nxt
    return vs[0]


def _tec_body(stamp_hbm, pe_hbm, table_hbm, gamma_hbm, beta_hbm, out_hbm,
              idx_v, rows_v, out_v, pe_v, g_v, b_v, isem, gsem, osem):
    wid = lax.axis_index("s") * _NC + lax.axis_index("c")
    pltpu.sync_copy(pe_hbm, pe_v)
    pltpu.sync_copy(gamma_hbm, g_v)
    pltpu.sync_copy(beta_hbm, b_v)
    b0 = wid * _BPW
    iota = lax.iota(jnp.int32, 16)
    gsp = [plsc.load_gather(g_v, [jnp.full((16,), d, jnp.int32)])
           for d in range(_D)]
    bsp = [plsc.load_gather(b_v, [jnp.full((16,), d, jnp.int32)])
           for d in range(_D)]

    def start_idx(l, b):
        off = l * _B + b0
        pltpu.async_copy(stamp_hbm.at[pl.ds(off, _BPW)], idx_v[b], isem[b])

    def wait_idx(b):
        pltpu.make_async_copy(stamp_hbm.at[pl.ds(0, _BPW)], idx_v[b],
                              isem[b]).wait()

    def start_gather(b):
        pltpu.async_copy(table_hbm.at[idx_v[b]], rows_v[b], gsem[b])

    def wait_gather(b):
        pltpu.make_async_copy(table_hbm.at[idx_v[b]], rows_v[b],
                              gsem[b]).wait()

    def start_out(l, b):
        for i in range(2):
            off = (l * 256 + i * 128 + wid * 4) * 1024
            pltpu.async_copy(out_v[b].at[pl.ds(i * 4096, 4096)],
                             out_hbm.at[pl.ds(off, 4096)], osem[b])

    def wait_out(b):
        for i in range(2):
            pltpu.make_async_copy(out_v[b].at[pl.ds(i * 4096, 4096)],
                                  out_hbm.at[pl.ds(0, 4096)], osem[b]).wait()

    def compute(l, rows, out):
        lbase = l * 256
        psp = [pe_v[pl.ds(lbase + d * 16, 16)] for d in range(_D)]

        def one_group(g):
            rowidx = g * 16 + iota
            gpart = (g // 8) * 1024 + (g % 8) * 16
            xs = []
            for d in range(_D):
                e = plsc.load_gather(rows, [rowidx,
                                            jnp.full((16,), d, jnp.int32)])
                xs.append(e + psp[d])
            mu = _tree_sum(xs) * (1.0 / _D)
            devs = [x - mu for x in xs]
            s2 = _tree_sum([dv * dv for dv in devs])
            r = _rsqrt(s2 * (1.0 / _D) + 1e-5)
            for d in range(_D):
                obase = gpart + (d // 8) * 4096 + (d % 8) * 128
                plsc.store_scatter(out, [obase + iota],
                                   devs[d] * (r * gsp[d]) + bsp[d])

        def group_body(h, carry):
            for q in range(4):
                one_group(4 * h + q)
            return carry

        lax.fori_loop(0, _NG // 4, group_body, 0)

    start_idx(0, 0)
    start_idx(1, 1)
    wait_idx(0)
    start_gather(0)

    def outer(i, carry):
        for b in range(2):
            c = 2 * i + b

            @pl.when(c + 1 < _L)
            def _():
                wait_idx(b ^ 1)
                start_gather(b ^ 1)

            wait_gather(b)

            @pl.when(c >= 2)
            def _():
                wait_out(b)

            compute(c, rows_v[b], out_v[b])
            start_out(c, b)

            @pl.when(c + 2 < _L)
            def _():
                start_idx(c + 2, b)
        return carry

    lax.fori_loop(0, _L // 2, outer, 0)
    wait_out(0)
    wait_out(1)


def kernel(stamp, table, ln_gamma, ln_beta):
    stamp_t = jnp.transpose(stamp).reshape(-1)
    pe_const = jnp.asarray(_PE_LANE)
    mesh = plsc.VectorSubcoreMesh(core_axis_name="c", subcore_axis_name="s")
    run = pl.kernel(
        _tec_body,
        compiler_params=pltpu.CompilerParams(
            needs_layout_passes=False, use_tc_tiling_on_sc=False),
        out_type=jax.ShapeDtypeStruct((_N * _D,), jnp.float32),
        mesh=mesh,
        scratch_types=[
            [pltpu.VMEM((_BPW,), jnp.int32)] * 2,
            [pltpu.VMEM((_BPW, _D), jnp.float32)] * 2,
            [pltpu.VMEM((_BPW * _D,), jnp.float32)] * 2,
            pltpu.VMEM((_L * _D * 16,), jnp.float32),
            pltpu.VMEM((_D,), jnp.float32),
            pltpu.VMEM((_D,), jnp.float32),
            [pltpu.SemaphoreType.DMA] * 2,
            [pltpu.SemaphoreType.DMA] * 2,
            [pltpu.SemaphoreType.DMA] * 2,
        ],
    )
    out = run(stamp_t, pe_const, table, ln_gamma, ln_beta)
    out = out.reshape(_L, 2, _B // 128, 8, 128)
    return jnp.transpose(out, (2, 4, 0, 1, 3)).reshape(_B, _L, _D)
